# Initial kernel scaffold; baseline (speedup 1.0000x reference)
import functools

import jax
import jax.numpy as jnp
from jax import lax
from jax.experimental import pallas as pl
from jax.experimental.pallas import tpu as pltpu
from jax.experimental.pallas import tpu_sc as plsc

_L = 16
_IDS_PER_ROW = 125
_ROWS_PER_CHUNK = 8
_SLOTS_PER_CHUNK = _ROWS_PER_CHUNK * _IDS_PER_ROW // 5


def _rsqrt_newton(x):
    i = lax.bitcast_convert_type(x, jnp.int32)
    i = jnp.int32(0x5F3759DF) - lax.shift_right_logical(i, 1)
    y = lax.bitcast_convert_type(i, jnp.float32)
    for _ in range(3):
        y = y * (1.5 - 0.5 * x * y * y)
    return y


def _make_sc_kernel(num_slots, seq_len, parts, dim, num_workers):
    rows_per_worker = num_slots * parts // _IDS_PER_ROW // num_workers
    slots_per_worker = num_slots // num_workers
    chunks = rows_per_worker // _ROWS_PER_CHUNK
    ncg = dim // _L

    mesh = plsc.VectorSubcoreMesh(core_axis_name="c", subcore_axis_name="s")

    @functools.partial(
        pl.kernel,
        out_type=jax.ShapeDtypeStruct((num_slots, dim), jnp.float32),
        mesh=mesh,
        scratch_types=[
            pltpu.VMEM((_ROWS_PER_CHUNK, _IDS_PER_ROW), jnp.int32),
            pltpu.VMEM((_ROWS_PER_CHUNK * _IDS_PER_ROW, dim), jnp.float32),
            pltpu.VMEM((_SLOTS_PER_CHUNK, dim), jnp.float32),
            pltpu.VMEM((seq_len, dim), jnp.float32),
            pltpu.VMEM((dim,), jnp.float32),
            pltpu.VMEM((dim,), jnp.float32),
            pltpu.SemaphoreType.DMA,
        ],
    )
    def body(ids_ref, table_ref, pos_ref, gam_ref, bet_ref, out_ref,
             idx_v, rows_v, out_v, pos_v, gam_v, bet_v, sem):
        nc = mesh.num_cores
        wid = lax.axis_index("s") * nc + lax.axis_index("c")
        wrow = wid * rows_per_worker
        wslot = wid * slots_per_worker

        pltpu.sync_copy(pos_ref.at[pl.ds(0, seq_len)], pos_v)
        pltpu.sync_copy(gam_ref, gam_v)
        pltpu.sync_copy(bet_ref, bet_v)
        gam = [gam_v[pl.ds(cg * _L, _L)] for cg in range(ncg)]
        bet = [bet_v[pl.ds(cg * _L, _L)] for cg in range(ncg)]
        inv_parts = jnp.float32(1.0 / parts)
        inv_dim = jnp.float32(1.0 / dim)

        def slot_body(i, _):
            p = lax.rem(i, seq_len)
            base = i * parts
            accs = []
            for cg in range(ncg):
                col = cg * _L
                a = rows_v[base, pl.ds(col, _L)]
                for k in range(1, parts):
                    a = a + rows_v[base + k, pl.ds(col, _L)]
                a = a * inv_parts + pos_v[p, pl.ds(col, _L)]
                accs.append(a)
            t = (accs[0] + accs[1]) + (accs[2] + accs[3])
            sq = (accs[0] * accs[0] + accs[1] * accs[1]) + (
                accs[2] * accs[2] + accs[3] * accs[3])
            mean = jnp.sum(t) * inv_dim
            var = jnp.sum(sq) * inv_dim - mean * mean
            inv = _rsqrt_newton(var + jnp.float32(1e-5))
            for cg in range(ncg):
                out_v[i, pl.ds(cg * _L, _L)] = (
                    (accs[cg] - mean) * inv * gam[cg] + bet[cg])
            return 0

        def chunk_body(c, _):
            jrow = wrow + c * _ROWS_PER_CHUNK
            pltpu.sync_copy(ids_ref.at[pl.ds(jrow, _ROWS_PER_CHUNK)], idx_v)
            handles = [
                pltpu.async_copy(
                    table_ref.at[idx_v.at[jj]],
                    rows_v.at[pl.ds(jj * _IDS_PER_ROW, _IDS_PER_ROW)],
                    sem)
                for jj in range(_ROWS_PER_CHUNK)
            ]
            for h in handles:
                h.wait()
            lax.fori_loop(0, _SLOTS_PER_CHUNK, slot_body, 0)
            pltpu.sync_copy(
                out_v, out_ref.at[pl.ds(wslot + c * _SLOTS_PER_CHUNK,
                                        _SLOTS_PER_CHUNK)])
            return 0

        lax.fori_loop(0, chunks, chunk_body, 0)

    return body


def kernel(part_ids, part_table, pos_table, ln_gamma, ln_beta):
    batch, seq_len, parts = part_ids.shape
    dim = part_table.shape[1]
    num_slots = batch * seq_len
    ids2d = part_ids.reshape(num_slots * parts // _IDS_PER_ROW, _IDS_PER_ROW)
    sc = _make_sc_kernel(num_slots, seq_len, parts, dim, 32)
    out = sc(ids2d, part_table, pos_table, ln_gamma, ln_beta)
    return out.reshape(batch, seq_len, dim)


# speedup vs baseline: 7.2694x; 7.2694x over previous
import functools

import jax
import jax.numpy as jnp
from jax import lax
from jax.experimental import pallas as pl
from jax.experimental.pallas import tpu as pltpu
from jax.experimental.pallas import tpu_sc as plsc

_L = 16
_IDS_PER_ROW = 125
_ROWS_PER_CHUNK = 8
_SLOTS_PER_CHUNK = _ROWS_PER_CHUNK * _IDS_PER_ROW // 5


def _rsqrt_newton(x):
    i = lax.bitcast_convert_type(x, jnp.int32)
    i = jnp.int32(0x5F3759DF) - lax.shift_right_logical(i, 1)
    y = lax.bitcast_convert_type(i, jnp.float32)
    for _ in range(3):
        y = y * (1.5 - 0.5 * x * y * y)
    return y


_GATHER_DNUMS = lax.GatherDimensionNumbers(
    offset_dims=(), collapsed_slice_dims=(0,), start_index_map=(0,))


def _lane_permute(v, idx):
    return lax.gather(v, idx[:, None], _GATHER_DNUMS, slice_sizes=(1,),
                      mode=lax.GatherScatterMode.PROMISE_IN_BOUNDS)


def _lane_sum(v, perm_idx):
    for idx in perm_idx:
        v = v + _lane_permute(v, idx)
    return v


def _make_sc_kernel(num_slots, seq_len, parts, dim, max_pos, num_workers):
    rows_per_worker = num_slots * parts // _IDS_PER_ROW // num_workers
    slots_per_worker = num_slots // num_workers
    chunks = rows_per_worker // _ROWS_PER_CHUNK
    ncg = dim // _L

    mesh = plsc.VectorSubcoreMesh(core_axis_name="c", subcore_axis_name="s")

    @functools.partial(
        pl.kernel,
        out_type=jax.ShapeDtypeStruct((num_slots, dim), jnp.float32),
        mesh=mesh,
        scratch_types=[
            pltpu.VMEM((_ROWS_PER_CHUNK, _IDS_PER_ROW), jnp.int32),
            pltpu.VMEM((_ROWS_PER_CHUNK * _IDS_PER_ROW, dim), jnp.float32),
            pltpu.VMEM((_SLOTS_PER_CHUNK, dim), jnp.float32),
            pltpu.VMEM((max_pos, dim), jnp.float32),
            pltpu.VMEM((dim,), jnp.float32),
            pltpu.VMEM((dim,), jnp.float32),
            pltpu.SemaphoreType.DMA,
        ],
        compiler_params=pltpu.CompilerParams(use_tc_tiling_on_sc=False),
    )
    def body(ids_ref, table_ref, pos_ref, gam_ref, bet_ref, out_ref,
             idx_v, rows_v, out_v, pos_v, gam_v, bet_v, sem):
        nc = mesh.num_cores
        wid = lax.axis_index("s") * nc + lax.axis_index("c")
        wrow = wid * rows_per_worker
        wslot = wid * slots_per_worker

        pltpu.sync_copy(pos_ref, pos_v)
        pltpu.sync_copy(gam_ref, gam_v)
        pltpu.sync_copy(bet_ref, bet_v)
        gam = [gam_v[pl.ds(cg * _L, _L)] for cg in range(ncg)]
        bet = [bet_v[pl.ds(cg * _L, _L)] for cg in range(ncg)]
        inv_parts = jnp.float32(1.0 / parts)
        inv_dim = jnp.float32(1.0 / dim)
        lanes = lax.iota(jnp.int32, _L)
        perm_idx = [lanes ^ jnp.int32(sh) for sh in (8, 4, 2, 1)]

        def slot_body(i, _):
            p = lax.rem(i, seq_len)
            base = i * parts
            accs = []
            for cg in range(ncg):
                col = cg * _L
                a = rows_v[base, pl.ds(col, _L)]
                for k in range(1, parts):
                    a = a + rows_v[base + k, pl.ds(col, _L)]
                a = a * inv_parts + pos_v[p, pl.ds(col, _L)]
                accs.append(a)
            t = (accs[0] + accs[1]) + (accs[2] + accs[3])
            sq = (accs[0] * accs[0] + accs[1] * accs[1]) + (
                accs[2] * accs[2] + accs[3] * accs[3])
            mean = _lane_sum(t, perm_idx) * inv_dim
            var = _lane_sum(sq, perm_idx) * inv_dim - mean * mean
            inv = _rsqrt_newton(var + jnp.float32(1e-5))
            for cg in range(ncg):
                out_v[i, pl.ds(cg * _L, _L)] = (
                    (accs[cg] - mean) * inv * gam[cg] + bet[cg])
            return 0

        def chunk_body(c, _):
            jrow = wrow + c * _ROWS_PER_CHUNK
            pltpu.sync_copy(ids_ref.at[pl.ds(jrow, _ROWS_PER_CHUNK)], idx_v)
            handles = [
                pltpu.async_copy(
                    table_ref.at[idx_v.at[jj]],
                    rows_v.at[pl.ds(jj * _IDS_PER_ROW, _IDS_PER_ROW)],
                    sem)
                for jj in range(_ROWS_PER_CHUNK)
            ]
            for h in handles:
                h.wait()
            lax.fori_loop(0, _SLOTS_PER_CHUNK, slot_body, 0)
            pltpu.sync_copy(
                out_v, out_ref.at[pl.ds(wslot + c * _SLOTS_PER_CHUNK,
                                        _SLOTS_PER_CHUNK)])
            return 0

        lax.fori_loop(0, chunks, chunk_body, 0)

    return body


def kernel(part_ids, part_table, pos_table, ln_gamma, ln_beta):
    batch, seq_len, parts = part_ids.shape
    dim = part_table.shape[1]
    num_slots = batch * seq_len
    ids2d = part_ids.reshape(num_slots * parts // _IDS_PER_ROW, _IDS_PER_ROW)
    sc = _make_sc_kernel(num_slots, seq_len, parts, dim, pos_table.shape[0], 32)
    out = sc(ids2d, part_table, pos_table, ln_gamma, ln_beta)
    return out.reshape(batch, seq_len, dim)


# speedup vs baseline: 10.7978x; 1.4854x over previous
import functools

import jax
import jax.numpy as jnp
from jax import lax
from jax.experimental import pallas as pl
from jax.experimental.pallas import tpu as pltpu
from jax.experimental.pallas import tpu_sc as plsc

_L = 16
_IDS_PER_ROW = 125
_ROWS_PER_CHUNK = 8
_SLOTS_PER_CHUNK = _ROWS_PER_CHUNK * _IDS_PER_ROW // 5


def _rsqrt_newton(x):
    i = lax.bitcast_convert_type(x, jnp.int32)
    i = jnp.int32(0x5F3759DF) - lax.shift_right_logical(i, 1)
    y = lax.bitcast_convert_type(i, jnp.float32)
    for _ in range(3):
        y = y * (1.5 - 0.5 * x * y * y)
    return y


_GATHER_DNUMS = lax.GatherDimensionNumbers(
    offset_dims=(), collapsed_slice_dims=(0,), start_index_map=(0,))


def _lane_permute(v, idx):
    return lax.gather(v, idx[:, None], _GATHER_DNUMS, slice_sizes=(1,),
                      mode=lax.GatherScatterMode.PROMISE_IN_BOUNDS)


def _lane_sum(v, perm_idx):
    for idx in perm_idx:
        v = v + _lane_permute(v, idx)
    return v


def _make_sc_kernel(num_slots, seq_len, parts, dim, max_pos, num_workers):
    rows_per_worker = num_slots * parts // _IDS_PER_ROW // num_workers
    slots_per_worker = num_slots // num_workers
    chunks = rows_per_worker // _ROWS_PER_CHUNK
    ncg = dim // _L

    mesh = plsc.VectorSubcoreMesh(core_axis_name="c", subcore_axis_name="s")

    @functools.partial(
        pl.kernel,
        out_type=jax.ShapeDtypeStruct((num_slots, dim), jnp.float32),
        mesh=mesh,
        scratch_types=[
            pltpu.VMEM((_ROWS_PER_CHUNK, _IDS_PER_ROW), jnp.int32),
            pltpu.VMEM((_ROWS_PER_CHUNK * _IDS_PER_ROW, dim), jnp.float32),
            pltpu.VMEM((_SLOTS_PER_CHUNK, dim), jnp.float32),
            pltpu.VMEM((max_pos, dim), jnp.float32),
            pltpu.VMEM((dim,), jnp.float32),
            pltpu.VMEM((dim,), jnp.float32),
            pltpu.SemaphoreType.DMA,
        ],
        compiler_params=pltpu.CompilerParams(use_tc_tiling_on_sc=False),
    )
    def body(ids_ref, table_ref, pos_ref, gam_ref, bet_ref, out_ref,
             idx_v, rows_v, out_v, pos_v, gam_v, bet_v, sem):
        nc = mesh.num_cores
        wid = lax.axis_index("s") * nc + lax.axis_index("c")
        wrow = wid * rows_per_worker
        wslot = wid * slots_per_worker

        pltpu.sync_copy(pos_ref, pos_v)
        pltpu.sync_copy(gam_ref, gam_v)
        pltpu.sync_copy(bet_ref, bet_v)
        gam = [gam_v[pl.ds(cg * _L, _L)] for cg in range(ncg)]
        bet = [bet_v[pl.ds(cg * _L, _L)] for cg in range(ncg)]
        inv_parts = jnp.float32(1.0 / parts)
        inv_dim = jnp.float32(1.0 / dim)
        lanes = lax.iota(jnp.int32, _L)
        perm_idx = [lanes ^ jnp.int32(sh) for sh in (8, 4, 2, 1)]

        def slot_body(i, _):
            p = lax.rem(i, seq_len)
            base = i * parts
            accs = []
            for cg in range(ncg):
                col = cg * _L
                a = rows_v[base, pl.ds(col, _L)]
                for k in range(1, parts):
                    a = a + rows_v[base + k, pl.ds(col, _L)]
                a = a * inv_parts + pos_v[p, pl.ds(col, _L)]
                accs.append(a)
            t = (accs[0] + accs[1]) + (accs[2] + accs[3])
            sq = (accs[0] * accs[0] + accs[1] * accs[1]) + (
                accs[2] * accs[2] + accs[3] * accs[3])
            mean = _lane_sum(t, perm_idx) * inv_dim
            var = _lane_sum(sq, perm_idx) * inv_dim - mean * mean
            inv = _rsqrt_newton(var + jnp.float32(1e-5))
            for cg in range(ncg):
                out_v[i, pl.ds(cg * _L, _L)] = (
                    (accs[cg] - mean) * inv * gam[cg] + bet[cg])
            return 0

        def chunk_body(c, _):
            jrow = wrow + c * _ROWS_PER_CHUNK
            pltpu.sync_copy(ids_ref.at[pl.ds(jrow, _ROWS_PER_CHUNK)], idx_v)
            handles = [
                pltpu.async_copy(
                    table_ref.at[idx_v.at[jj]],
                    rows_v.at[pl.ds(jj * _IDS_PER_ROW, _IDS_PER_ROW)],
                    sem)
                for jj in range(_ROWS_PER_CHUNK)
            ]
            for h in handles:
                h.wait()

            @plsc.parallel_loop(0, _SLOTS_PER_CHUNK, unroll=4)
            def _(i):
                slot_body(i, 0)
            pltpu.sync_copy(
                out_v, out_ref.at[pl.ds(wslot + c * _SLOTS_PER_CHUNK,
                                        _SLOTS_PER_CHUNK)])
            return 0

        lax.fori_loop(0, chunks, chunk_body, 0)

    return body


def kernel(part_ids, part_table, pos_table, ln_gamma, ln_beta):
    batch, seq_len, parts = part_ids.shape
    dim = part_table.shape[1]
    num_slots = batch * seq_len
    ids2d = part_ids.reshape(num_slots * parts // _IDS_PER_ROW, _IDS_PER_ROW)
    sc = _make_sc_kernel(num_slots, seq_len, parts, dim, pos_table.shape[0], 32)
    out = sc(ids2d, part_table, pos_table, ln_gamma, ln_beta)
    return out.reshape(batch, seq_len, dim)


# speedup vs baseline: 14.4594x; 1.3391x over previous
import functools

import jax
import jax.numpy as jnp
from jax import lax
from jax.experimental import pallas as pl
from jax.experimental.pallas import tpu as pltpu
from jax.experimental.pallas import tpu_sc as plsc

_L = 16
_IDS_PER_ROW = 125
_ROWS_PER_CHUNK = 4
_IDS_PER_CHUNK = _ROWS_PER_CHUNK * _IDS_PER_ROW
_SLOTS_PER_CHUNK = _IDS_PER_CHUNK // 5


def _rsqrt_newton(x):
    i = lax.bitcast_convert_type(x, jnp.int32)
    i = jnp.int32(0x5F3759DF) - lax.shift_right_logical(i, 1)
    y = lax.bitcast_convert_type(i, jnp.float32)
    for _ in range(3):
        y = y * (1.5 - 0.5 * x * y * y)
    return y


_GATHER_DNUMS = lax.GatherDimensionNumbers(
    offset_dims=(), collapsed_slice_dims=(0,), start_index_map=(0,))


def _lane_permute(v, idx):
    return lax.gather(v, idx[:, None], _GATHER_DNUMS, slice_sizes=(1,),
                      mode=lax.GatherScatterMode.PROMISE_IN_BOUNDS)


def _lane_sum(v, perm_idx):
    for idx in perm_idx:
        v = v + _lane_permute(v, idx)
    return v


def _make_sc_kernel(num_slots, seq_len, parts, dim, max_pos, num_workers):
    total_chunks = num_slots // _SLOTS_PER_CHUNK
    chunks = total_chunks // num_workers
    assert chunks % 2 == 0
    ncg = dim // _L

    mesh = plsc.VectorSubcoreMesh(core_axis_name="c", subcore_axis_name="s")

    @functools.partial(
        pl.kernel,
        out_type=jax.ShapeDtypeStruct((total_chunks, _SLOTS_PER_CHUNK, dim),
                                      jnp.float32),
        mesh=mesh,
        scratch_types=[
            pltpu.VMEM((2, _ROWS_PER_CHUNK, _IDS_PER_ROW), jnp.int32),
            pltpu.VMEM((2, _IDS_PER_CHUNK, dim), jnp.float32),
            pltpu.VMEM((2, _SLOTS_PER_CHUNK, dim), jnp.float32),
            pltpu.VMEM((max_pos, dim), jnp.float32),
            pltpu.VMEM((dim,), jnp.float32),
            pltpu.VMEM((dim,), jnp.float32),
            pltpu.SemaphoreType.DMA,
            pltpu.SemaphoreType.DMA,
            pltpu.SemaphoreType.DMA,
            pltpu.SemaphoreType.DMA,
            pltpu.SemaphoreType.DMA,
        ],
        compiler_params=pltpu.CompilerParams(use_tc_tiling_on_sc=False),
    )
    def body(ids_ref, table_ref, pos_ref, gam_ref, bet_ref, out_ref,
             idx_v, rows_v, out_v, pos_v, gam_v, bet_v,
             gsem0, gsem1, isem0, isem1, osem):
        nc = mesh.num_cores
        wid = lax.axis_index("s") * nc + lax.axis_index("c")
        wchunk = wid * chunks
        gsems = (gsem0, gsem1)
        isems = (isem0, isem1)

        pltpu.sync_copy(pos_ref, pos_v)
        pltpu.sync_copy(gam_ref, gam_v)
        pltpu.sync_copy(bet_ref, bet_v)
        gam = [gam_v[pl.ds(cg * _L, _L)] for cg in range(ncg)]
        bet = [bet_v[pl.ds(cg * _L, _L)] for cg in range(ncg)]
        inv_parts = jnp.float32(1.0 / parts)
        inv_dim = jnp.float32(1.0 / dim)
        lanes = lax.iota(jnp.int32, _L)
        perm_idx = [lanes ^ jnp.int32(sh) for sh in (8, 4, 2, 1)]

        def fire_gathers(b):
            return [
                pltpu.async_copy(
                    table_ref.at[idx_v.at[b, jj]],
                    rows_v.at[b, pl.ds(jj * _IDS_PER_ROW, _IDS_PER_ROW)],
                    gsems[b])
                for jj in range(_ROWS_PER_CHUNK)
            ]

        def compute(b):
            @plsc.parallel_loop(0, _SLOTS_PER_CHUNK, unroll=4)
            def _(i):
                p = lax.rem(i, seq_len)
                base = i * parts
                accs = []
                for cg in range(ncg):
                    col = cg * _L
                    a = rows_v[b, base, pl.ds(col, _L)]
                    for k in range(1, parts):
                        a = a + rows_v[b, base + k, pl.ds(col, _L)]
                    a = a * inv_parts + pos_v[p, pl.ds(col, _L)]
                    accs.append(a)
                t = (accs[0] + accs[1]) + (accs[2] + accs[3])
                sq = (accs[0] * accs[0] + accs[1] * accs[1]) + (
                    accs[2] * accs[2] + accs[3] * accs[3])
                mean = _lane_sum(t, perm_idx) * inv_dim
                var = _lane_sum(sq, perm_idx) * inv_dim - mean * mean
                inv = _rsqrt_newton(var + jnp.float32(1e-5))
                for cg in range(ncg):
                    out_v[b, i, pl.ds(cg * _L, _L)] = (
                        (accs[cg] - mean) * inv * gam[cg] + bet[cg])

        pltpu.sync_copy(ids_ref.at[wchunk], idx_v.at[0])
        fire_gathers(0)
        pltpu.async_copy(ids_ref.at[wchunk + 1], idx_v.at[1], isems[1])

        def pair_body(c0, _):
            for b in range(2):
                c = c0 + b
                nb = 1 - b
                @pl.when(c + 1 < chunks)
                def _():
                    pltpu.make_async_copy(
                        ids_ref.at[wchunk + c + 1], idx_v.at[nb],
                        isems[nb]).wait()
                    fire_gathers(nb)
                for jj in range(_ROWS_PER_CHUNK):
                    pltpu.make_async_copy(
                        table_ref.at[idx_v.at[b, jj]],
                        rows_v.at[b, pl.ds(jj * _IDS_PER_ROW, _IDS_PER_ROW)],
                        gsems[b]).wait()

                @pl.when(c + 2 < chunks)
                def _():
                    pltpu.async_copy(ids_ref.at[wchunk + c + 2],
                                     idx_v.at[b], isems[b])
                compute(b)

                @pl.when(c > 0)
                def _():
                    pltpu.make_async_copy(
                        out_v.at[nb], out_ref.at[wchunk + c - 1],
                        osem).wait()
                pltpu.async_copy(out_v.at[b], out_ref.at[wchunk + c], osem)
            return 0

        lax.fori_loop(0, chunks // 2, lambda j, x: pair_body(j * 2, x), 0)
        pltpu.make_async_copy(
            out_v.at[1], out_ref.at[wchunk + chunks - 1], osem).wait()

    return body


def kernel(part_ids, part_table, pos_table, ln_gamma, ln_beta):
    batch, seq_len, parts = part_ids.shape
    dim = part_table.shape[1]
    num_slots = batch * seq_len
    total_chunks = num_slots // _SLOTS_PER_CHUNK
    ids3d = part_ids.reshape(total_chunks, _ROWS_PER_CHUNK, _IDS_PER_ROW)
    sc = _make_sc_kernel(num_slots, seq_len, parts, dim,
                         pos_table.shape[0], 32)
    out = sc(ids3d, part_table, pos_table, ln_gamma, ln_beta)
    return out.reshape(batch, seq_len, dim)
